# baseline (speedup 1.0000x reference)
import functools

import jax
import jax.numpy as jnp
from jax import lax
from jax.experimental import pallas as pl
from jax.experimental.pallas import tpu as pltpu
from jax.experimental.pallas import tpu_sc as plsc

_N = 10000
_E = 320000
_NC = 2
_NS = 16
_CH = 80
_NB = 5


def _make_agg(feature_split):
    rows_per_tile = 624
    tail_rows = _N - _NS * rows_per_tile
    tail_r0 = _NS * rows_per_tile
    n_workers = _NS if feature_split else _NC * _NS
    n_chunks = _E // (n_workers * _CH)
    mesh = plsc.VectorSubcoreMesh(core_axis_name="c", subcore_axis_name="s")

    @functools.partial(
        pl.kernel,
        out_type=jax.ShapeDtypeStruct((_NC, _N, 64), jnp.float32),
        mesh=mesh,
        scratch_types=[
            pltpu.VMEM((n_chunks, _CH), jnp.int32),
            pltpu.VMEM((n_chunks, _CH), jnp.int32),
            pltpu.VMEM((8, _CH), jnp.int32),
        ] + [pltpu.VMEM((_CH, 64), jnp.float32) for _ in range(_NB)]
          + [pltpu.VMEM_SHARED((_N, 64), jnp.float32)]
          + [pltpu.SemaphoreType.DMA for _ in range(2 * _NB)],
        compiler_params=pltpu.CompilerParams(use_tc_tiling_on_sc=False,
                                             disable_bounds_checks=True),
    )
    def agg(x_hbm, src_hbm, dst_hbm, out_hbm, srcs, dsts, sidx, *rest):
        bufs = rest[:_NB]
        acc = rest[_NB]
        gsems = rest[_NB + 1:2 * _NB + 1]
        ssems = rest[2 * _NB + 1:]
        c = lax.axis_index("c")
        s = lax.axis_index("s")
        table = x_hbm
        r0 = pl.multiple_of(s * rows_per_tile, 8)
        if feature_split:
            half = n_chunks // 2
            pltpu.sync_copy(src_hbm.at[2 * s], srcs.at[pl.ds(0, half)])
            pltpu.sync_copy(src_hbm.at[2 * s + 1], srcs.at[pl.ds(half, half)])
            pltpu.sync_copy(dst_hbm.at[2 * s], dsts.at[pl.ds(0, half)])
            pltpu.sync_copy(dst_hbm.at[2 * s + 1], dsts.at[pl.ds(half, half)])
        else:
            w = c * _NS + s
            pltpu.sync_copy(src_hbm.at[w], srcs)
            pltpu.sync_copy(dst_hbm.at[w], dsts)

        if feature_split:
            def xform(j):
                for k in range(_CH // 16):
                    v = srcs[j, pl.ds(16 * k, 16)]
                    srcs[j, pl.ds(16 * k, 16)] = 2 * v + c

            sizes = [_CH] * 7 + [rows_per_tile - 7 * _CH]

            def sfill(t):
                for k in range(_CH // 16):
                    sidx[t, pl.ds(16 * k, 16)] = (
                        2 * (r0 + _CH * t + 16 * k +
                             lax.iota(jnp.int32, 16)) + c)

            def sgather(t, k):
                return pltpu.async_copy(
                    table.at[sidx.at[t, pl.ds(0, sizes[t])]],
                    bufs[k].at[pl.ds(0, sizes[t])], gsems[k])

            def sgwait(t, k):
                pltpu.make_async_copy(
                    table.at[sidx.at[t, pl.ds(0, sizes[t])]],
                    bufs[k].at[pl.ds(0, sizes[t])], gsems[k]).wait()

            for t in range(_NB):
                sfill(t)
                sgather(t, t)
            for t in range(len(sizes)):
                k = t % _NB
                sgwait(t, k)
                pltpu.sync_copy(bufs[k].at[pl.ds(0, sizes[t])],
                                acc.at[pl.ds(r0 + _CH * t, sizes[t])])
                if t + _NB < len(sizes):
                    sfill(t + _NB)
                    sgather(t + _NB, k)

            @pl.when(s == 0)
            def _seed_tail():
                for k in range(_CH // 16):
                    sidx[0, pl.ds(16 * k, 16)] = (
                        2 * (tail_r0 + 16 * k + lax.iota(jnp.int32, 16)) + c)
                pltpu.async_copy(table.at[sidx.at[0, pl.ds(0, tail_rows)]],
                                 bufs[0].at[pl.ds(0, tail_rows)],
                                 gsems[0]).wait()
                pltpu.sync_copy(bufs[0].at[pl.ds(0, tail_rows)],
                                acc.at[pl.ds(tail_r0, tail_rows)])
        else:
            pltpu.sync_copy(table.at[pl.ds(r0, rows_per_tile)],
                            acc.at[pl.ds(r0, rows_per_tile)])

            @pl.when(s == 0)
            def _seed_tail():
                pltpu.sync_copy(table.at[pl.ds(tail_r0, tail_rows)],
                                acc.at[pl.ds(tail_r0, tail_rows)])

        plsc.subcore_barrier()

        def gather(j, k):
            return pltpu.async_copy(table.at[srcs.at[j]], bufs[k], gsems[k])

        def gwait(j, k):
            pltpu.make_async_copy(table.at[srcs.at[j]], bufs[k],
                                  gsems[k]).wait()

        def scat(j, k):
            return pltpu.async_copy(bufs[k], acc.at[dsts.at[j]], ssems[k],
                                    add=True)

        def swait(j, k):
            pltpu.make_async_copy(bufs[k], acc.at[dsts.at[j]],
                                  ssems[k]).wait()

        for k in range(_NB):
            if feature_split:
                xform(k)
            gather(k, k)

        def body(i, carry):
            j = _NB * i
            for k in range(_NB):
                gwait(j + k, k)
                scat(j + k, k)
            for k in range(_NB):
                swait(j + k, k)

                @pl.when(j + k + _NB < n_chunks)
                def _refill(j=j, k=k):
                    if feature_split:
                        xform(j + k + _NB)
                    gather(j + k + _NB, k)
            return carry

        lax.fori_loop(0, n_chunks // _NB, body, 0)
        plsc.subcore_barrier()
        pltpu.sync_copy(acc.at[pl.ds(r0, rows_per_tile)],
                        out_hbm.at[c, pl.ds(r0, rows_per_tile)])

        @pl.when(s == 0)
        def _write_tail():
            pltpu.sync_copy(acc.at[pl.ds(tail_r0, tail_rows)],
                            out_hbm.at[c, pl.ds(tail_r0, tail_rows)])

    return agg


_agg_l1 = _make_agg(feature_split=True)
_agg_l2 = _make_agg(feature_split=False)


def _halves_bn_relu(te, to, g, bt):
    mu = 0.5 * (jnp.mean(te, axis=0) + jnp.mean(to, axis=0))
    m2 = 0.5 * (jnp.mean(te * te, axis=0) + jnp.mean(to * to, axis=0))
    var = m2 - mu * mu
    a = g * lax.rsqrt(var + 1e-5)
    b = bt - mu * a
    return jnp.maximum(te * a + b, 0.0), jnp.maximum(to * a + b, 0.0)


def _dot(x, w):
    return jnp.dot(x, w, preferred_element_type=jnp.float32)


def _mlp1_body(a_ref, w1_ref, b1_ref, g_ref, bt_ref, w2_ref, b2_ref, o_ref):
    a0 = a_ref[0]
    a1 = a_ref[1]
    w1a = w1_ref[:64, :]
    w1b = w1_ref[64:, :]
    b1 = b1_ref[...]
    te = _dot(a0[:, :64], w1a) + _dot(a1[:, :64], w1b) + b1
    to = _dot(a0[:, 64:], w1a) + _dot(a1[:, 64:], w1b) + b1
    te, to = _halves_bn_relu(te, to, g_ref[...], bt_ref[...])
    w2 = w2_ref[...]
    b2 = b2_ref[...]
    o_ref[:, :64] = jnp.maximum(_dot(te, w2) + b2, 0.0)
    o_ref[:, 64:] = jnp.maximum(_dot(to, w2) + b2, 0.0)


def _mlp2_body(x_ref, a_ref, w1_ref, b1_ref, g_ref, bt_ref, w2_ref, b2_ref,
               fcw_ref, fcb_ref, emb_ref, out_ref):
    hp = a_ref[0] + a_ref[1] - x_ref[...]
    w1 = w1_ref[...]
    b1 = b1_ref[...]
    te = _dot(hp[:, :64], w1) + b1
    to = _dot(hp[:, 64:], w1) + b1
    te, to = _halves_bn_relu(te, to, g_ref[...], bt_ref[...])
    w2 = w2_ref[...]
    b2 = b2_ref[...]
    h2e = jnp.maximum(_dot(te, w2) + b2, 0.0)
    h2o = jnp.maximum(_dot(to, w2) + b2, 0.0)
    emb_ref[:, :32] = h2e
    emb_ref[:, 32:] = h2o
    fcw = fcw_ref[...]
    fcb = fcb_ref[...]
    out_ref[:, :64] = _dot(h2e, fcw) + fcb
    out_ref[:, 64:] = _dot(h2o, fcw) + fcb


def kernel(x, edge_index, l1_w1, l1_b1, l1_bn_g, l1_bn_b, l1_w2, l1_b2,
           l2_w1, l2_b1, l2_bn_g, l2_bn_b, l2_w2, l2_b2, fc_w, fc_b):
    src = edge_index[0].reshape(_NC * _NS, _E // (_NC * _NS * _CH), _CH)
    dst = edge_index[1].reshape(_NC * _NS, _E // (_NC * _NS * _CH), _CH)

    agg1 = _agg_l1(x.reshape(2 * _N, 64), src, dst)
    h1p = pl.pallas_call(
        _mlp1_body,
        out_shape=jax.ShapeDtypeStruct((_N // 2, 128), jnp.float32),
    )(agg1.reshape(_NC, _N // 2, 128), l1_w1, l1_b1, l1_bn_g, l1_bn_b,
      l1_w2, l1_b2)

    agg2 = _agg_l2(h1p.reshape(_N, 64), src, dst)
    emb_p, out_p = pl.pallas_call(
        _mlp2_body,
        out_shape=(
            jax.ShapeDtypeStruct((_N // 2, 64), jnp.float32),
            jax.ShapeDtypeStruct((_N // 2, 128), jnp.float32),
        ),
    )(h1p, agg2.reshape(_NC, _N // 2, 128), l2_w1, l2_b1, l2_bn_g, l2_bn_b,
      l2_w2, l2_b2, fc_w, fc_b)

    return emb_p.reshape(_N, 32), out_p.reshape(_N, 64)
